# baseline (speedup 1.0000x reference)
import functools

import jax
import jax.numpy as jnp
from jax.experimental import pallas as pl


def _chamfer_block_kernel(n_i, a_ref, b_ref, min_a_ref, min_b_ref):
    i = pl.program_id(1)
    d2 = jax.lax.dot_general(
        a_ref[0], b_ref[0], (((1,), (1,)), ((), ())),
        preferred_element_type=jnp.float32,
    )
    min_a_ref[0, 0, :] = jnp.sqrt(jnp.maximum(jnp.min(d2, axis=1), 0.0))
    colmin = jnp.min(d2, axis=0)

    @pl.when(i == 0)
    def _init():
        min_b_ref[0, 0, :] = colmin

    @pl.when(i > 0)
    def _acc():
        min_b_ref[0, 0, :] = jnp.minimum(min_b_ref[0, 0, :], colmin)

    @pl.when(i == n_i - 1)
    def _fin():
        min_b_ref[0, 0, :] = jnp.sqrt(jnp.maximum(min_b_ref[0, 0, :], 0.0))


def kernel(A, B):
    bt, n, d = A.shape
    m = B.shape[1]
    bi = 512
    n_i = n // bi
    da = 128

    a2 = jnp.sum(A * A, axis=-1, keepdims=True)
    b2 = jnp.sum(B * B, axis=-1, keepdims=True)
    ones_a = jnp.ones((bt, n, 1), jnp.float32)
    ones_b = jnp.ones((bt, m, 1), jnp.float32)
    pad_a = jnp.zeros((bt, n, da - d - 2), jnp.float32)
    pad_b = jnp.zeros((bt, m, da - d - 2), jnp.float32)
    a_aug = jnp.concatenate([A, a2, ones_a, pad_a], axis=-1)
    b_aug = jnp.concatenate([-2.0 * B, ones_b, b2, pad_b], axis=-1)

    min_a, min_b = pl.pallas_call(
        functools.partial(_chamfer_block_kernel, n_i),
        grid=(bt, n_i),
        in_specs=[
            pl.BlockSpec((1, bi, da), lambda b, i: (b, i, 0)),
            pl.BlockSpec((1, m, da), lambda b, i: (b, 0, 0)),
        ],
        out_specs=[
            pl.BlockSpec((1, 1, bi), lambda b, i: (b * n_i + i, 0, 0)),
            pl.BlockSpec((1, 1, m), lambda b, i: (b, 0, 0)),
        ],
        out_shape=[
            jax.ShapeDtypeStruct((bt * n_i, 1, bi), jnp.float32),
            jax.ShapeDtypeStruct((bt, 1, m), jnp.float32),
        ],
    )(a_aug, b_aug)
    min_a = min_a.reshape(bt, n)
    min_b = min_b.reshape(bt, m)
    chamfer = jnp.mean(min_a, axis=1) + jnp.mean(min_b, axis=1)
    return jnp.mean(chamfer) / 12.8


# speedup vs baseline: 1.2355x; 1.2355x over previous
import functools

import jax
import jax.numpy as jnp
from jax.experimental import pallas as pl


def _chamfer_batch_kernel(n_i, bi, a_ref, bs_ref, a2r_ref, a2c_ref, b2_ref,
                          min_a_ref, min_b_ref):
    bm = bs_ref[0]
    b2 = b2_ref[0, 0, :]

    def step(i, colmin):
        a = a_ref[0, pl.ds(i * bi, bi), :]
        t = jax.lax.dot_general(
            a, bm, (((1,), (1,)), ((), ())),
            preferred_element_type=jnp.float32,
        )
        u = t + b2[None, :]
        rowmin = jnp.min(u, axis=1) + a2r_ref[0, 0, pl.ds(i * bi, bi)]
        min_a_ref[0, 0, pl.ds(i * bi, bi)] = jnp.sqrt(jnp.maximum(rowmin, 0.0))
        w = t + a2c_ref[0, pl.ds(i * bi, bi), :]
        return jnp.minimum(colmin, jnp.min(w, axis=0))

    init = jnp.full((b2.shape[0],), jnp.inf, jnp.float32)
    colmin = jax.lax.fori_loop(0, n_i, step, init)
    min_b_ref[0, 0, :] = jnp.sqrt(jnp.maximum(colmin + b2, 0.0))


def kernel(A, B):
    bt, n, d = A.shape
    m = B.shape[1]
    bi = 512
    n_i = n // bi

    a2 = jnp.sum(A * A, axis=-1)
    b2 = jnp.sum(B * B, axis=-1)
    bs = -2.0 * B
    a2r = a2.reshape(bt, 1, n)
    a2c = a2.reshape(bt, n, 1)
    b2r = b2.reshape(bt, 1, m)

    min_a, min_b = pl.pallas_call(
        functools.partial(_chamfer_batch_kernel, n_i, bi),
        grid=(bt,),
        in_specs=[
            pl.BlockSpec((1, n, d), lambda b: (b, 0, 0)),
            pl.BlockSpec((1, m, d), lambda b: (b, 0, 0)),
            pl.BlockSpec((1, 1, n), lambda b: (b, 0, 0)),
            pl.BlockSpec((1, n, 1), lambda b: (b, 0, 0)),
            pl.BlockSpec((1, 1, m), lambda b: (b, 0, 0)),
        ],
        out_specs=[
            pl.BlockSpec((1, 1, n), lambda b: (b, 0, 0)),
            pl.BlockSpec((1, 1, m), lambda b: (b, 0, 0)),
        ],
        out_shape=[
            jax.ShapeDtypeStruct((bt, 1, n), jnp.float32),
            jax.ShapeDtypeStruct((bt, 1, m), jnp.float32),
        ],
    )(A, bs, a2r, a2c, b2r)
    min_a = min_a.reshape(bt, n)
    min_b = min_b.reshape(bt, m)
    chamfer = jnp.mean(min_a, axis=1) + jnp.mean(min_b, axis=1)
    return jnp.mean(chamfer) / 12.8


# speedup vs baseline: 1.7692x; 1.4320x over previous
import functools

import jax
import jax.numpy as jnp
from jax.experimental import pallas as pl


def _chamfer_batch_kernel(n_i, bi, a_ref, b_ref, min_a_ref, min_b_ref):
    bm = b_ref[0]
    m = bm.shape[0]
    b2 = jnp.sum(bm * bm, axis=1, keepdims=True)
    ones_b = jnp.ones((m, 1), jnp.float32)
    bm_aug = jnp.concatenate([-2.0 * bm, ones_b, b2], axis=1)

    def step(i, colmin):
        a = a_ref[0, pl.ds(i * bi, bi), :]
        a2 = jnp.sum(a * a, axis=1, keepdims=True)
        ones_a = jnp.ones((bi, 1), jnp.float32)
        a_aug = jnp.concatenate([a, a2, ones_a], axis=1)
        d2 = jax.lax.dot_general(
            a_aug, bm_aug, (((1,), (1,)), ((), ())),
            preferred_element_type=jnp.float32,
        )
        rowmin = jnp.min(d2, axis=1, keepdims=True)
        min_a_ref[0, pl.ds(i * bi, bi), :] = jnp.sqrt(jnp.maximum(rowmin, 0.0))
        return jnp.minimum(colmin, jnp.min(d2, axis=0))

    init = jnp.full((m,), jnp.inf, jnp.float32)
    colmin = jax.lax.fori_loop(0, n_i, step, init)
    min_b_ref[0, 0, :] = jnp.sqrt(jnp.maximum(colmin, 0.0))


def kernel(A, B):
    bt, n, d = A.shape
    m = B.shape[1]
    bi = 512
    n_i = n // bi

    min_a, min_b = pl.pallas_call(
        functools.partial(_chamfer_batch_kernel, n_i, bi),
        grid=(bt,),
        in_specs=[
            pl.BlockSpec((1, n, d), lambda b: (b, 0, 0)),
            pl.BlockSpec((1, m, d), lambda b: (b, 0, 0)),
        ],
        out_specs=[
            pl.BlockSpec((1, n, 1), lambda b: (b, 0, 0)),
            pl.BlockSpec((1, 1, m), lambda b: (b, 0, 0)),
        ],
        out_shape=[
            jax.ShapeDtypeStruct((bt, n, 1), jnp.float32),
            jax.ShapeDtypeStruct((bt, 1, m), jnp.float32),
        ],
    )(A, B)
    min_a = min_a.reshape(bt, n)
    min_b = min_b.reshape(bt, m)
    chamfer = jnp.mean(min_a, axis=1) + jnp.mean(min_b, axis=1)
    return jnp.mean(chamfer) / 12.8
